# Initial kernel scaffold; baseline (speedup 1.0000x reference)
import functools

import jax
import jax.numpy as jnp
from jax import lax
from jax.experimental import pallas as pl
from jax.experimental.pallas import tpu as pltpu
from jax.experimental.pallas import tpu_sc as plsc

ROWS_PER_CHUNK = 128


def _sc_gather(tables, idx_arrays, B):
    info = plsc.get_sparse_core_info()
    NC, NS = info.num_cores, info.num_subcores
    NW = NC * NS
    b_per_w = B // NW
    n_chunks = b_per_w // ROWS_PER_CHUNK
    n_tab = len(tables)

    mesh = plsc.VectorSubcoreMesh(core_axis_name="c", subcore_axis_name="s")

    idx2 = [a.astype(jnp.int32).reshape(B // ROWS_PER_CHUNK, ROWS_PER_CHUNK)
            for a in idx_arrays]

    out_type = [jax.ShapeDtypeStruct((B, 16), jnp.float32) for _ in range(n_tab)]
    scratch_types = (
        [pltpu.VMEM((n_chunks, ROWS_PER_CHUNK), jnp.int32) for _ in range(n_tab)]
        + [pltpu.VMEM((b_per_w, 16), jnp.float32) for _ in range(n_tab)]
        + [pltpu.SemaphoreType.DMA]
    )

    @functools.partial(
        pl.kernel, mesh=mesh, out_type=out_type, scratch_types=scratch_types,
    )
    def k(*refs):
        tabs = refs[:n_tab]
        idxs = refs[n_tab:2 * n_tab]
        outs = refs[2 * n_tab:3 * n_tab]
        idx_v = refs[3 * n_tab:4 * n_tab]
        rows_v = refs[4 * n_tab:5 * n_tab]
        sem = refs[5 * n_tab]

        wid = lax.axis_index("s") * NC + lax.axis_index("c")
        base = wid * b_per_w
        r0 = wid * n_chunks

        for t in range(n_tab):
            pltpu.sync_copy(idxs[t].at[pl.ds(r0, n_chunks)], idx_v[t])
        copies = []
        for t in range(n_tab):
            for j in range(n_chunks):
                copies.append(pltpu.async_copy(
                    tabs[t].at[idx_v[t].at[j]],
                    rows_v[t].at[pl.ds(j * ROWS_PER_CHUNK, ROWS_PER_CHUNK)],
                    sem))
        for c in copies:
            c.wait()
        for t in range(n_tab):
            pltpu.sync_copy(rows_v[t], outs[t].at[pl.ds(base, b_per_w)])

    return list(k(*tables, *idx2))


def _mlp_body(ctx_ref, bat_ref, bowl_ref, bt_ref, bwt_ref, ven_ref,
              wc1_ref, bc1_ref, wc2_ref, bc2_ref,
              w1x_ref, b1_ref, w2_ref, b2_ref, w3_ref, b3_ref, out_ref):
    f32 = jnp.float32
    ctx = ctx_ref[...]
    wc1 = wc1_ref[...]
    h = ctx[:, 0:1] * wc1[0:1, :] + ctx[:, 1:2] * wc1[1:2, :] + bc1_ref[...]
    h = jnp.maximum(h, 0.0)
    cv = jnp.maximum(
        jnp.dot(h, wc2_ref[...], preferred_element_type=f32) + bc2_ref[...], 0.0)
    w1x = w1x_ref[...]
    h1 = (jnp.dot(bat_ref[...], w1x[0:16], preferred_element_type=f32)
          + jnp.dot(bowl_ref[...], w1x[16:32], preferred_element_type=f32)
          + jnp.dot(bt_ref[...], w1x[32:48], preferred_element_type=f32)
          + jnp.dot(bwt_ref[...], w1x[48:64], preferred_element_type=f32)
          + jnp.dot(ven_ref[...], w1x[64:80], preferred_element_type=f32)
          + jnp.dot(cv, w1x[80:96], preferred_element_type=f32)
          + b1_ref[...])
    h1 = jnp.maximum(h1, 0.0)
    h2 = jnp.maximum(
        jnp.dot(h1, w2_ref[...], preferred_element_type=f32) + b2_ref[...], 0.0)
    out_ref[...] = jnp.dot(h2, w3_ref[...], preferred_element_type=f32) + b3_ref[...]


def kernel(striker_ids, bowler_ids, bat_team_ids, bowl_team_ids, venue_ids,
           context, bat_emb, bowl_emb, team_emb, venue_emb,
           w_c1, b_c1, w_c2, b_c2, w1, b1, w2, b2, w3, b3):
    B = striker_ids.shape[0]

    team_pad = jnp.pad(team_emb, ((0, 0), (0, 8)))
    venue_pad = jnp.pad(venue_emb, ((0, 0), (0, 8)))

    bat_g, bowl_g, bt_g, bwt_g, ven_g = _sc_gather(
        [bat_emb, bowl_emb, team_pad, team_pad, venue_pad],
        [striker_ids, bowler_ids, bat_team_ids, bowl_team_ids, venue_ids],
        B)

    pad8 = lambda m: jnp.pad(m, ((0, 8), (0, 0)))
    w1x = jnp.concatenate([
        w1[0:32],
        pad8(w1[32:40]), pad8(w1[40:48]), pad8(w1[48:56]),
        w1[56:72],
    ], axis=0)

    BK = 2048
    grid = (B // BK,)
    row_spec16 = pl.BlockSpec((BK, 16), lambda i: (i, 0))
    full = lambda s: pl.BlockSpec(s, lambda i: tuple(0 for _ in s))

    out = pl.pallas_call(
        _mlp_body,
        grid=grid,
        in_specs=[
            pl.BlockSpec((BK, context.shape[1]), lambda i: (i, 0)),
            row_spec16, row_spec16, row_spec16, row_spec16, row_spec16,
            full(w_c1.shape), full((1, b_c1.shape[0])),
            full(w_c2.shape), full((1, b_c2.shape[0])),
            full((96, 128)), full((1, b1.shape[0])),
            full(w2.shape), full((1, b2.shape[0])),
            full(w3.shape), full((1, b3.shape[0])),
        ],
        out_specs=pl.BlockSpec((BK, w3.shape[1]), lambda i: (i, 0)),
        out_shape=jax.ShapeDtypeStruct((B, w3.shape[1]), jnp.float32),
    )(context, bat_g, bowl_g, bt_g, bwt_g, ven_g,
      w_c1, b_c1.reshape(1, -1), w_c2, b_c2.reshape(1, -1),
      w1x, b1.reshape(1, -1), w2, b2.reshape(1, -1), w3, b3.reshape(1, -1))
    return out


# speedup vs baseline: 1.8487x; 1.8487x over previous
import functools

import jax
import jax.numpy as jnp
from jax import lax
from jax.experimental import pallas as pl
from jax.experimental.pallas import tpu as pltpu
from jax.experimental.pallas import tpu_sc as plsc

ROWS_PER_CHUNK = 128


def _sc_gather(tables, idx_arrays, B):
    info = plsc.get_sparse_core_info()
    NC, NS = info.num_cores, info.num_subcores
    NW = NC * NS
    b_per_w = B // NW
    n_chunks = b_per_w // ROWS_PER_CHUNK
    n_tab = len(tables)

    mesh = plsc.VectorSubcoreMesh(core_axis_name="c", subcore_axis_name="s")

    idx2 = [a.astype(jnp.int32).reshape(B // ROWS_PER_CHUNK, ROWS_PER_CHUNK)
            for a in idx_arrays]

    out_type = [jax.ShapeDtypeStruct((B, 16), jnp.float32) for _ in range(n_tab)]
    scratch_types = (
        [pltpu.VMEM((n_chunks, ROWS_PER_CHUNK), jnp.int32) for _ in range(n_tab)]
        + [pltpu.VMEM((b_per_w, 16), jnp.float32) for _ in range(n_tab)]
        + [pltpu.SemaphoreType.DMA]
    )

    @functools.partial(
        pl.kernel, mesh=mesh, out_type=out_type, scratch_types=scratch_types,
        compiler_params=pltpu.CompilerParams(use_tc_tiling_on_sc=False),
    )
    def k(*refs):
        tabs = refs[:n_tab]
        idxs = refs[n_tab:2 * n_tab]
        outs = refs[2 * n_tab:3 * n_tab]
        idx_v = refs[3 * n_tab:4 * n_tab]
        rows_v = refs[4 * n_tab:5 * n_tab]
        sem = refs[5 * n_tab]

        wid = lax.axis_index("s") * NC + lax.axis_index("c")
        base = wid * b_per_w
        r0 = wid * n_chunks

        for t in range(n_tab):
            pltpu.sync_copy(idxs[t].at[pl.ds(r0, n_chunks)], idx_v[t])
        copies = []
        for t in range(n_tab):
            for j in range(n_chunks):
                copies.append(pltpu.async_copy(
                    tabs[t].at[idx_v[t].at[j]],
                    rows_v[t].at[pl.ds(j * ROWS_PER_CHUNK, ROWS_PER_CHUNK)],
                    sem))
        for c in copies:
            c.wait()
        for t in range(n_tab):
            pltpu.sync_copy(rows_v[t], outs[t].at[pl.ds(base, b_per_w)])

    return list(k(*tables, *idx2))


def _mlp_body(ctx_ref, bat_ref, bowl_ref, bt_ref, bwt_ref, ven_ref,
              wc1_ref, bc1_ref, wc2_ref, bc2_ref,
              w1x_ref, b1_ref, w2_ref, b2_ref, w3_ref, b3_ref, out_ref):
    f32 = jnp.float32
    ctx = ctx_ref[...]
    wc1 = wc1_ref[...]
    h = ctx[:, 0:1] * wc1[0:1, :] + ctx[:, 1:2] * wc1[1:2, :] + bc1_ref[...]
    h = jnp.maximum(h, 0.0)
    cv = jnp.maximum(
        jnp.dot(h, wc2_ref[...], preferred_element_type=f32) + bc2_ref[...], 0.0)
    w1x = w1x_ref[...]
    h1 = (jnp.dot(bat_ref[...], w1x[0:16], preferred_element_type=f32)
          + jnp.dot(bowl_ref[...], w1x[16:32], preferred_element_type=f32)
          + jnp.dot(bt_ref[...], w1x[32:48], preferred_element_type=f32)
          + jnp.dot(bwt_ref[...], w1x[48:64], preferred_element_type=f32)
          + jnp.dot(ven_ref[...], w1x[64:80], preferred_element_type=f32)
          + jnp.dot(cv, w1x[80:96], preferred_element_type=f32)
          + b1_ref[...])
    h1 = jnp.maximum(h1, 0.0)
    h2 = jnp.maximum(
        jnp.dot(h1, w2_ref[...], preferred_element_type=f32) + b2_ref[...], 0.0)
    out_ref[...] = jnp.dot(h2, w3_ref[...], preferred_element_type=f32) + b3_ref[...]


def kernel(striker_ids, bowler_ids, bat_team_ids, bowl_team_ids, venue_ids,
           context, bat_emb, bowl_emb, team_emb, venue_emb,
           w_c1, b_c1, w_c2, b_c2, w1, b1, w2, b2, w3, b3):
    B = striker_ids.shape[0]

    team_pad = jnp.pad(team_emb, ((0, 0), (0, 8)))
    venue_pad = jnp.pad(venue_emb, ((0, 0), (0, 8)))

    bat_g, bowl_g, bt_g, bwt_g, ven_g = _sc_gather(
        [bat_emb, bowl_emb, team_pad, team_pad, venue_pad],
        [striker_ids, bowler_ids, bat_team_ids, bowl_team_ids, venue_ids],
        B)

    pad8 = lambda m: jnp.pad(m, ((0, 8), (0, 0)))
    w1x = jnp.concatenate([
        w1[0:32],
        pad8(w1[32:40]), pad8(w1[40:48]), pad8(w1[48:56]),
        w1[56:72],
    ], axis=0)

    BK = 2048
    grid = (B // BK,)
    row_spec16 = pl.BlockSpec((BK, 16), lambda i: (i, 0))
    full = lambda s: pl.BlockSpec(s, lambda i: tuple(0 for _ in s))

    out = pl.pallas_call(
        _mlp_body,
        grid=grid,
        in_specs=[
            pl.BlockSpec((BK, context.shape[1]), lambda i: (i, 0)),
            row_spec16, row_spec16, row_spec16, row_spec16, row_spec16,
            full(w_c1.shape), full((1, b_c1.shape[0])),
            full(w_c2.shape), full((1, b_c2.shape[0])),
            full((96, 128)), full((1, b1.shape[0])),
            full(w2.shape), full((1, b2.shape[0])),
            full(w3.shape), full((1, b3.shape[0])),
        ],
        out_specs=pl.BlockSpec((BK, w3.shape[1]), lambda i: (i, 0)),
        out_shape=jax.ShapeDtypeStruct((B, w3.shape[1]), jnp.float32),
    )(context, bat_g, bowl_g, bt_g, bwt_g, ven_g,
      w_c1, b_c1.reshape(1, -1), w_c2, b_c2.reshape(1, -1),
      w1x, b1.reshape(1, -1), w2, b2.reshape(1, -1), w3, b3.reshape(1, -1))
    return out


# speedup vs baseline: 2.4622x; 1.3319x over previous
import functools

import jax
import jax.numpy as jnp
from jax import lax
from jax.experimental import pallas as pl
from jax.experimental.pallas import tpu as pltpu
from jax.experimental.pallas import tpu_sc as plsc

ROWS_PER_CHUNK = 128


def _sc_gather_combined(tables, idx_arrays, B):
    info = plsc.get_sparse_core_info()
    NC, NS = info.num_cores, info.num_subcores
    NW = NC * NS
    b_per_w = B // NW
    n_chunks = b_per_w // ROWS_PER_CHUNK
    n_tab = len(tables)

    mesh = plsc.VectorSubcoreMesh(core_axis_name="c", subcore_axis_name="s")

    idx2 = [a.astype(jnp.int32).reshape(B // ROWS_PER_CHUNK, ROWS_PER_CHUNK)
            for a in idx_arrays]

    out_type = jax.ShapeDtypeStruct((B, 128), jnp.float32)
    scratch_types = (
        [pltpu.VMEM((n_chunks, ROWS_PER_CHUNK), jnp.int32) for _ in range(n_tab)]
        + [pltpu.VMEM((b_per_w, 16), jnp.float32) for _ in range(n_tab)]
        + [pltpu.SemaphoreType.DMA]
    )

    @functools.partial(
        pl.kernel, mesh=mesh, out_type=out_type, scratch_types=scratch_types,
        compiler_params=pltpu.CompilerParams(use_tc_tiling_on_sc=False),
    )
    def k(*refs):
        tabs = refs[:n_tab]
        idxs = refs[n_tab:2 * n_tab]
        out = refs[2 * n_tab]
        idx_v = refs[2 * n_tab + 1:3 * n_tab + 1]
        rows_v = refs[3 * n_tab + 1:4 * n_tab + 1]
        sem = refs[4 * n_tab + 1]

        wid = lax.axis_index("s") * NC + lax.axis_index("c")
        base = wid * b_per_w
        r0 = wid * n_chunks

        for t in range(n_tab):
            pltpu.sync_copy(idxs[t].at[pl.ds(r0, n_chunks)], idx_v[t])
        copies = []
        for t in range(n_tab):
            for j in range(n_chunks):
                copies.append(pltpu.async_copy(
                    tabs[t].at[idx_v[t].at[j]],
                    rows_v[t].at[pl.ds(j * ROWS_PER_CHUNK, ROWS_PER_CHUNK)],
                    sem))
        for c in copies:
            c.wait()
        for t in range(8):
            pltpu.sync_copy(
                rows_v[t % n_tab],
                out.at[pl.ds(base, b_per_w), pl.ds(t * 16, 16)])

    return k(*tables, *idx2)


def _mlp_body(g_ref, ctxt_ref,
              wc1t_ref, bc1_ref, wc2_ref, bc2_ref,
              w1g_ref, w1c_ref, b1_ref, w2_ref, b2_ref, w3_ref, b3_ref,
              out_ref):
    f32 = jnp.float32
    dn_rt = (((0,), (1,)), ((), ()))
    dn_ll = (((0,), (0,)), ((), ()))
    ctx_t = ctxt_ref[...]
    wc1t = wc1t_ref[...]
    h_t = jnp.maximum(
        wc1t[:, 0:1] * ctx_t[0:1, :] + wc1t[:, 1:2] * ctx_t[1:2, :]
        + bc1_ref[...], 0.0)
    cv_t = jnp.maximum(
        lax.dot_general(wc2_ref[...], h_t, dn_ll, preferred_element_type=f32)
        + bc2_ref[...], 0.0)
    h1_t = jnp.maximum(
        lax.dot_general(w1g_ref[...], g_ref[...], dn_rt,
                        preferred_element_type=f32)
        + lax.dot_general(w1c_ref[...], cv_t, dn_ll,
                          preferred_element_type=f32)
        + b1_ref[...], 0.0)
    h2_t = jnp.maximum(
        lax.dot_general(w2_ref[...], h1_t, dn_ll, preferred_element_type=f32)
        + b2_ref[...], 0.0)
    out_ref[...] = (
        lax.dot_general(w3_ref[...], h2_t, dn_ll, preferred_element_type=f32)
        + b3_ref[...])


def kernel(striker_ids, bowler_ids, bat_team_ids, bowl_team_ids, venue_ids,
           context, bat_emb, bowl_emb, team_emb, venue_emb,
           w_c1, b_c1, w_c2, b_c2, w1, b1, w2, b2, w3, b3):
    B = striker_ids.shape[0]

    bat_t = bat_emb.reshape(-1).reshape(bat_emb.shape)
    bowl_t = bowl_emb.reshape(-1).reshape(bowl_emb.shape)
    team_pad = jnp.pad(team_emb, ((0, 0), (0, 8)))
    team_t = team_pad.reshape(-1).reshape(team_pad.shape)
    venue_pad = jnp.pad(venue_emb, ((0, 0), (0, 8)))
    venue_t = venue_pad.reshape(-1).reshape(venue_pad.shape)

    g = _sc_gather_combined(
        [bat_t, bowl_t, team_t, team_t, venue_t],
        [striker_ids, bowler_ids, bat_team_ids, bowl_team_ids, venue_ids],
        B)

    pad8 = lambda m: jnp.pad(m, ((0, 8), (0, 0)))
    w1g = jnp.concatenate([
        w1[0:32],
        pad8(w1[32:40]), pad8(w1[40:48]), pad8(w1[48:56]),
        jnp.zeros((48, 128), jnp.float32),
    ], axis=0)
    w1c = w1[56:72]

    BK = 2048
    grid = (B // BK,)
    full = lambda s: pl.BlockSpec(s, lambda i: tuple(0 for _ in s))

    out_t = pl.pallas_call(
        _mlp_body,
        grid=grid,
        in_specs=[
            pl.BlockSpec((BK, 128), lambda i: (i, 0)),
            pl.BlockSpec((context.shape[1], BK), lambda i: (0, i)),
            full((32, 2)), full((32, 1)),
            full((32, 16)), full((16, 1)),
            full((128, 128)), full((16, 128)), full((128, 1)),
            full((128, 64)), full((64, 1)),
            full((64, 10)), full((10, 1)),
        ],
        out_specs=pl.BlockSpec((10, BK), lambda i: (0, i)),
        out_shape=jax.ShapeDtypeStruct((10, B), jnp.float32),
    )(g, context.T,
      w_c1.T, b_c1.reshape(-1, 1), w_c2, b_c2.reshape(-1, 1),
      w1g, w1c, b1.reshape(-1, 1),
      w2, b2.reshape(-1, 1), w3, b3.reshape(-1, 1))
    return out_t.T


# speedup vs baseline: 2.7276x; 1.1078x over previous
import functools

import jax
import jax.numpy as jnp
from jax import lax
from jax.experimental import pallas as pl
from jax.experimental.pallas import tpu as pltpu
from jax.experimental.pallas import tpu_sc as plsc

ROWS_PER_CHUNK = 128


def _sc_gather_combined(tables, idx_arrays, B):
    info = plsc.get_sparse_core_info()
    NC, NS = info.num_cores, info.num_subcores
    NW = NC * NS
    b_per_w = B // NW
    n_chunks = b_per_w // ROWS_PER_CHUNK
    n_tab = len(tables)

    mesh = plsc.VectorSubcoreMesh(core_axis_name="c", subcore_axis_name="s")

    idx2 = [a.astype(jnp.int32).reshape(B // ROWS_PER_CHUNK, ROWS_PER_CHUNK)
            for a in idx_arrays]

    out_type = jax.ShapeDtypeStruct((B, 128), jnp.float32)
    scratch_types = (
        [pltpu.VMEM((n_chunks, ROWS_PER_CHUNK), jnp.int32) for _ in range(n_tab)]
        + [pltpu.VMEM((b_per_w, 16), jnp.float32) for _ in range(n_tab)]
        + [pltpu.SemaphoreType.DMA]
    )

    @functools.partial(
        pl.kernel, mesh=mesh, out_type=out_type, scratch_types=scratch_types,
        compiler_params=pltpu.CompilerParams(use_tc_tiling_on_sc=False),
    )
    def k(*refs):
        tabs = refs[:n_tab]
        idxs = refs[n_tab:2 * n_tab]
        out = refs[2 * n_tab]
        idx_v = refs[2 * n_tab + 1:3 * n_tab + 1]
        rows_v = refs[3 * n_tab + 1:4 * n_tab + 1]
        sem = refs[4 * n_tab + 1]

        wid = lax.axis_index("s") * NC + lax.axis_index("c")
        base = wid * b_per_w
        r0 = wid * n_chunks

        for t in range(n_tab):
            pltpu.sync_copy(idxs[t].at[pl.ds(r0, n_chunks)], idx_v[t])
        copies = []
        for t in range(n_tab):
            for j in range(n_chunks):
                copies.append(pltpu.async_copy(
                    tabs[t].at[idx_v[t].at[j]],
                    rows_v[t].at[pl.ds(j * ROWS_PER_CHUNK, ROWS_PER_CHUNK)],
                    sem))
        for c in copies:
            c.wait()
        for t in range(8):
            pltpu.sync_copy(
                rows_v[t % n_tab],
                out.at[pl.ds(base, b_per_w), pl.ds(t * 16, 16)])

    return k(*tables, *idx2)


_FOLD_W = 2048


def _fold_body(x1_ref, x2_ref, y1_ref, y2_ref):
    for x_ref, y_ref in ((x1_ref, y1_ref), (x2_ref, y2_ref)):
        xt = x_ref[...].T.reshape(_FOLD_W // 8, 8, 16)
        y_ref[...] = jnp.concatenate(
            [xt[:, s, :] for s in range(8)], axis=1)


def _fold_tables(t1, t2):
    n = t1.shape[1]
    grid = (pl.cdiv(n, _FOLD_W),)
    in_spec = pl.BlockSpec((16, _FOLD_W), lambda i: (0, i))
    out_spec = pl.BlockSpec((_FOLD_W // 8, 128), lambda i: (i, 0))
    y1, y2 = pl.pallas_call(
        _fold_body,
        grid=grid,
        in_specs=[in_spec, in_spec],
        out_specs=[out_spec, out_spec],
        out_shape=[jax.ShapeDtypeStruct((pl.cdiv(n, 8), 128), jnp.float32)] * 2,
    )(t1, t2)
    return y1.reshape(-1), y2.reshape(-1)


def _mlp_body(g_ref, ctxt_ref,
              wc1t_ref, bc1_ref, wc2_ref, bc2_ref,
              w1g_ref, w1c_ref, b1_ref, w2_ref, b2_ref, w3_ref, b3_ref,
              out_ref):
    f32 = jnp.float32
    dn_rt = (((0,), (1,)), ((), ()))
    dn_ll = (((0,), (0,)), ((), ()))
    ctx_t = ctxt_ref[...]
    wc1t = wc1t_ref[...]
    h_t = jnp.maximum(
        wc1t[:, 0:1] * ctx_t[0:1, :] + wc1t[:, 1:2] * ctx_t[1:2, :]
        + bc1_ref[...], 0.0)
    cv_t = jnp.maximum(
        lax.dot_general(wc2_ref[...], h_t, dn_ll, preferred_element_type=f32)
        + bc2_ref[...], 0.0)
    h1_t = jnp.maximum(
        lax.dot_general(w1g_ref[...], g_ref[...], dn_rt,
                        preferred_element_type=f32)
        + lax.dot_general(w1c_ref[...], cv_t, dn_ll,
                          preferred_element_type=f32)
        + b1_ref[...], 0.0)
    h2_t = jnp.maximum(
        lax.dot_general(w2_ref[...], h1_t, dn_ll, preferred_element_type=f32)
        + b2_ref[...], 0.0)
    out_ref[...] = (
        lax.dot_general(w3_ref[...], h2_t, dn_ll, preferred_element_type=f32)
        + b3_ref[...])


def kernel(striker_ids, bowler_ids, bat_team_ids, bowl_team_ids, venue_ids,
           context, bat_emb, bowl_emb, team_emb, venue_emb,
           w_c1, b_c1, w_c2, b_c2, w1, b1, w2, b2, w3, b3):
    B = striker_ids.shape[0]

    bat_lin, bowl_lin = _fold_tables(bat_emb.T, bowl_emb.T)
    bat_t = bat_lin.reshape(bat_emb.shape)
    bowl_t = bowl_lin.reshape(bowl_emb.shape)
    team_pad = jnp.pad(team_emb, ((0, 0), (0, 8)))
    team_t = team_pad.reshape(-1).reshape(team_pad.shape)
    venue_pad = jnp.pad(venue_emb, ((0, 0), (0, 8)))
    venue_t = venue_pad.reshape(-1).reshape(venue_pad.shape)

    g = _sc_gather_combined(
        [bat_t, bowl_t, team_t, team_t, venue_t],
        [striker_ids, bowler_ids, bat_team_ids, bowl_team_ids, venue_ids],
        B)

    pad8 = lambda m: jnp.pad(m, ((0, 8), (0, 0)))
    w1g = jnp.concatenate([
        w1[0:32],
        pad8(w1[32:40]), pad8(w1[40:48]), pad8(w1[48:56]),
        jnp.zeros((48, 128), jnp.float32),
    ], axis=0)
    w1c = w1[56:72]

    BK = 2048
    grid = (B // BK,)
    full = lambda s: pl.BlockSpec(s, lambda i: tuple(0 for _ in s))

    out_t = pl.pallas_call(
        _mlp_body,
        grid=grid,
        in_specs=[
            pl.BlockSpec((BK, 128), lambda i: (i, 0)),
            pl.BlockSpec((context.shape[1], BK), lambda i: (0, i)),
            full((32, 2)), full((32, 1)),
            full((32, 16)), full((16, 1)),
            full((128, 128)), full((16, 128)), full((128, 1)),
            full((128, 64)), full((64, 1)),
            full((64, 10)), full((10, 1)),
        ],
        out_specs=pl.BlockSpec((10, BK), lambda i: (0, i)),
        out_shape=jax.ShapeDtypeStruct((10, B), jnp.float32),
    )(g, context.T,
      w_c1.T, b_c1.reshape(-1, 1), w_c2, b_c2.reshape(-1, 1),
      w1g, w1c, b1.reshape(-1, 1),
      w2, b2.reshape(-1, 1), w3, b3.reshape(-1, 1))
    return out_t.T


# speedup vs baseline: 3.7628x; 1.3795x over previous
import functools

import jax
import jax.numpy as jnp
from jax import lax
from jax.experimental import pallas as pl
from jax.experimental.pallas import tpu as pltpu
from jax.experimental.pallas import tpu_sc as plsc

ROWS_PER_CHUNK = 128
_TR_UNROLL = 8


def _sc_gather_combined(big_fm, small_tables, idx_arrays, B):
    info = plsc.get_sparse_core_info()
    NC, NS = info.num_cores, info.num_subcores
    NW = NC * NS
    b_per_w = B // NW
    n_chunks = b_per_w // ROWS_PER_CHUNK
    n_big = len(big_fm)
    n_small = len(small_tables)
    n_tab = n_big + n_small
    GP = b_per_w + 9

    mesh = plsc.VectorSubcoreMesh(core_axis_name="c", subcore_axis_name="s")

    idx2 = [a.astype(jnp.int32).reshape(B // ROWS_PER_CHUNK, ROWS_PER_CHUNK)
            for a in idx_arrays]

    out_type = jax.ShapeDtypeStruct((B, 128), jnp.float32)
    scratch_types = (
        [pltpu.VMEM((n_chunks, ROWS_PER_CHUNK), jnp.int32) for _ in range(n_tab)]
        + [pltpu.VMEM((16, GP), jnp.float32) for _ in range(n_big)]
        + [pltpu.VMEM((b_per_w, 16), jnp.float32) for _ in range(n_tab)]
        + [pltpu.SemaphoreType.DMA]
    )

    @functools.partial(
        pl.kernel, mesh=mesh, out_type=out_type, scratch_types=scratch_types,
        compiler_params=pltpu.CompilerParams(
            use_tc_tiling_on_sc=False, needs_layout_passes=False),
    )
    def k(*refs):
        tabs = refs[:n_tab]
        idxs = refs[n_tab:2 * n_tab]
        out = refs[2 * n_tab]
        a = 2 * n_tab + 1
        idx_v = refs[a:a + n_tab]
        gf = refs[a + n_tab:a + n_tab + n_big]
        rows_v = refs[a + n_tab + n_big:a + 2 * n_tab + n_big]
        sem = refs[a + 2 * n_tab + n_big]

        wid = lax.axis_index("s") * NC + lax.axis_index("c")
        base = wid * b_per_w
        r0 = wid * n_chunks
        rows16 = lax.iota(jnp.int32, 16)
        rows_gp = rows16 * GP

        for t in range(n_tab):
            pltpu.sync_copy(idxs[t].at[pl.ds(r0, n_chunks)], idx_v[t])
        copies = []
        for t in range(n_big):
            for f in range(16):
                for j in range(n_chunks):
                    copies.append(pltpu.async_copy(
                        tabs[t].at[f].at[idx_v[t].at[j]],
                        gf[t].at[f, pl.ds(j * ROWS_PER_CHUNK,
                                          ROWS_PER_CHUNK)],
                        sem))
        for t in range(n_big, n_tab):
            for j in range(n_chunks):
                copies.append(pltpu.async_copy(
                    tabs[t].at[idx_v[t].at[j]],
                    rows_v[t].at[pl.ds(j * ROWS_PER_CHUNK, ROWS_PER_CHUNK)],
                    sem))
        for c in copies:
            c.wait()

        for t in range(n_big):
            gf_t = gf[t]
            rows_t = rows_v[t]

            @plsc.parallel_loop(0, b_per_w, unroll=_TR_UNROLL)
            def _(j, gf_t=gf_t, rows_t=rows_t):
                vals = plsc.load_gather(gf_t, [rows16, rows16 * 0 + j])
                rows_t[j] = vals

        for t in range(n_tab):
            pltpu.sync_copy(
                rows_v[t],
                out.at[pl.ds(base, b_per_w), pl.ds(t * 16, 16)])

    return k(*big_fm, *small_tables, *idx2)


def _mlp_body(g_ref, ctxt_ref,
              wc1t_ref, bc1_ref, wc2_ref, bc2_ref,
              w1g_ref, w1c_ref, b1_ref, w2_ref, b2_ref, w3_ref, b3_ref,
              out_ref):
    f32 = jnp.float32
    dn_rt = (((0,), (1,)), ((), ()))
    dn_ll = (((0,), (0,)), ((), ()))
    ctx_t = ctxt_ref[...]
    wc1t = wc1t_ref[...]
    h_t = jnp.maximum(
        wc1t[:, 0:1] * ctx_t[0:1, :] + wc1t[:, 1:2] * ctx_t[1:2, :]
        + bc1_ref[...], 0.0)
    cv_t = jnp.maximum(
        lax.dot_general(wc2_ref[...], h_t, dn_ll, preferred_element_type=f32)
        + bc2_ref[...], 0.0)
    gs = g_ref[...][:, 0:80]
    h1_t = jnp.maximum(
        lax.dot_general(w1g_ref[...], gs, dn_rt, preferred_element_type=f32)
        + lax.dot_general(w1c_ref[...], cv_t, dn_ll,
                          preferred_element_type=f32)
        + b1_ref[...], 0.0)
    h2_t = jnp.maximum(
        lax.dot_general(w2_ref[...], h1_t, dn_ll, preferred_element_type=f32)
        + b2_ref[...], 0.0)
    out_ref[...] = (
        lax.dot_general(w3_ref[...], h2_t, dn_ll, preferred_element_type=f32)
        + b3_ref[...])


def kernel(striker_ids, bowler_ids, bat_team_ids, bowl_team_ids, venue_ids,
           context, bat_emb, bowl_emb, team_emb, venue_emb,
           w_c1, b_c1, w_c2, b_c2, w1, b1, w2, b2, w3, b3):
    B = striker_ids.shape[0]
    V = bat_emb.shape[0]

    bat_fm = bat_emb.T.reshape(-1).reshape(16, V)
    bowl_fm = bowl_emb.T.reshape(-1).reshape(16, V)
    team_pad = jnp.pad(team_emb, ((0, 0), (0, 8)))
    venue_pad = jnp.pad(venue_emb, ((0, 0), (0, 8)))

    g = _sc_gather_combined(
        [bat_fm, bowl_fm],
        [team_pad, team_pad, venue_pad],
        [striker_ids, bowler_ids, bat_team_ids, bowl_team_ids, venue_ids],
        B)

    pad8 = lambda m: jnp.pad(m, ((0, 8), (0, 0)))
    w1g = jnp.concatenate([
        w1[0:32],
        pad8(w1[32:40]), pad8(w1[40:48]), pad8(w1[48:56]),
    ], axis=0)
    w1c = w1[56:72]

    BK = 2048
    grid = (B // BK,)
    full = lambda s: pl.BlockSpec(s, lambda i: tuple(0 for _ in s))

    out_t = pl.pallas_call(
        _mlp_body,
        grid=grid,
        in_specs=[
            pl.BlockSpec((BK, 128), lambda i: (i, 0)),
            pl.BlockSpec((context.shape[1], BK), lambda i: (0, i)),
            full((32, 2)), full((32, 1)),
            full((32, 16)), full((16, 1)),
            full((80, 128)), full((16, 128)), full((128, 1)),
            full((128, 64)), full((64, 1)),
            full((64, 10)), full((10, 1)),
        ],
        out_specs=pl.BlockSpec((10, BK), lambda i: (0, i)),
        out_shape=jax.ShapeDtypeStruct((10, B), jnp.float32),
    )(g, context.T,
      w_c1.T, b_c1.reshape(-1, 1), w_c2, b_c2.reshape(-1, 1),
      w1g, w1c, b1.reshape(-1, 1),
      w2, b2.reshape(-1, 1), w3, b3.reshape(-1, 1))
    return out_t.T
